# baseline (speedup 1.0000x reference)
import functools

import jax
import jax.numpy as jnp
from jax import lax
from jax.experimental import pallas as pl
from jax.experimental.pallas import tpu as pltpu
from jax.experimental.pallas import tpu_sc as plsc

NC = 2
NS = 16
NW = NC * NS
L = 16
EMBED = 32
LANES = 128
NBUF = 8
BATCH = 16384
B_PER_W = BATCH // NW
WVLEN = 4 * L + L


def _sc_body(uidx_hbm, bidx_hbm, ut_hbm, bt_hbm, wb_hbm, out_hbm,
             uiv, biv, ustg, bstg, wv, outv, *sems):
    wid = lax.axis_index("s") * NC + lax.axis_index("c")
    base = wid * B_PER_W
    usems = sems[:NBUF]
    bsems = sems[NBUF:]

    pltpu.sync_copy(uidx_hbm.at[wid], uiv)
    pltpu.sync_copy(bidx_hbm.at[wid], biv)
    pltpu.sync_copy(wb_hbm, wv)

    wu0 = wv[pl.ds(0, L)]
    wu1 = wv[pl.ds(L, L)]
    wb0 = wv[pl.ds(2 * L, L)]
    wb1 = wv[pl.ds(3 * L, L)]
    bias16 = wv[pl.ds(4 * L, L)]
    d_lo = lax.iota(jnp.int32, L)
    d_hi = d_lo + L
    zero16 = jnp.zeros((L,), jnp.int32)

    def lane_of(k):
        start = (k // L) * L
        return start, k - start

    def col_scalar(iv, k):
        start, lane = lane_of(k)
        v = iv[pl.ds(start, L)]
        tcol = jnp.where(d_lo == lane, lax.shift_right_logical(v, 7), zero16)
        return pl.multiple_of(lax.reduce_max(tcol, axes=(0,)) * LANES, LANES)

    def lane_bcast(iv, k):
        start, lane = lane_of(k)
        v = iv[pl.ds(start, L)] & (LANES - 1)
        return v[jnp.full((L,), lane, jnp.int32)]

    def issue(k, p):
        pltpu.async_copy(ut_hbm.at[:, pl.ds(col_scalar(uiv, k), LANES)],
                         ustg.at[p], usems[p])
        pltpu.async_copy(bt_hbm.at[:, pl.ds(col_scalar(biv, k), LANES)],
                         bstg.at[p], bsems[p])

    def wait_slot(p):
        pltpu.make_async_copy(ut_hbm.at[:, pl.ds(0, LANES)],
                              ustg.at[p], usems[p]).wait()
        pltpu.make_async_copy(bt_hbm.at[:, pl.ds(0, LANES)],
                              bstg.at[p], bsems[p]).wait()

    def compute(j, p, vacc):
        clu = lane_bcast(uiv, j)
        clb = lane_bcast(biv, j)
        us = ustg.at[p]
        bs = bstg.at[p]
        acc = plsc.load_gather(us, [d_lo, clu]) * wu0 \
            + plsc.load_gather(us, [d_hi, clu]) * wu1 \
            + plsc.load_gather(bs, [d_lo, clb]) * wb0 \
            + plsc.load_gather(bs, [d_hi, clb]) * wb1 \
            + bias16
        s = lax.reduce_sum(acc, axes=(0,))
        _, lane = lane_of(j)
        return jnp.where(d_lo == lane, s, vacc)

    for k in range(NBUF):
        issue(k, k)

    def block(kb, vacc):
        for p in range(NBUF):
            k = kb * NBUF + p
            wait_slot(p)
            vacc = compute(k, p, vacc)

            @pl.when(k + NBUF < B_PER_W)
            def _():
                issue(k + NBUF, p)

        @pl.when(lax.rem(kb, L // NBUF) == L // NBUF - 1)
        def _():
            outv[pl.ds((kb // (L // NBUF)) * L, L)] = vacc
        return vacc

    lax.fori_loop(0, B_PER_W // NBUF, block, jnp.zeros((L,), jnp.float32))
    pltpu.sync_copy(outv, out_hbm.at[pl.ds(base, B_PER_W)])


def kernel(users, books, user_table, book_table, W, b):
    batch = users.shape[0]
    users_i = users.astype(jnp.int32)
    books_i = books.astype(jnp.int32)
    perm = jnp.argsort(users_i)
    users_i = users_i[perm]
    books_i = books_i[perm]
    uidx = users_i.reshape(NW, B_PER_W)
    bidx = books_i.reshape(NW, B_PER_W)
    ut_t = user_table.T
    bt_t = book_table.T
    wb = jnp.concatenate(
        [W.reshape(2 * EMBED).astype(jnp.float32),
         jnp.broadcast_to(b.reshape(1) / L, (L,))])

    mesh = plsc.VectorSubcoreMesh(core_axis_name="c", subcore_axis_name="s")
    fn = functools.partial(
        pl.kernel,
        out_type=jax.ShapeDtypeStruct((batch,), jnp.float32),
        mesh=mesh,
        scratch_types=[
            pltpu.VMEM((B_PER_W,), jnp.int32),
            pltpu.VMEM((B_PER_W,), jnp.int32),
            pltpu.VMEM((NBUF, EMBED, LANES), jnp.float32),
            pltpu.VMEM((NBUF, EMBED, LANES), jnp.float32),
            pltpu.VMEM((WVLEN,), jnp.float32),
            pltpu.VMEM((B_PER_W,), jnp.float32),
        ] + [pltpu.SemaphoreType.DMA] * (2 * NBUF),
        compiler_params=pltpu.CompilerParams(
            needs_layout_passes=False, use_tc_tiling_on_sc=True),
    )(_sc_body)
    out = fn(uidx, bidx, ut_t, bt_t, wb)
    out = jnp.zeros((batch,), jnp.float32).at[perm].set(out)
    return out.reshape(batch, 1)


# speedup vs baseline: 1.3529x; 1.3529x over previous
import functools

import jax
import jax.numpy as jnp
from jax import lax
from jax.experimental import pallas as pl
from jax.experimental.pallas import tpu as pltpu
from jax.experimental.pallas import tpu_sc as plsc

NC = 2
NS = 16
NW = NC * NS
L = 16
EMBED = 32
LANES = 128
NBUF = 8
BATCH = 16384
B_PER_W = BATCH // NW
WVLEN = 4 * L + L


def _sc_body(uidx_hbm, bidx_hbm, ut_hbm, bt_hbm, wb_hbm, out_hbm,
             uiv, biv, ustg, bstg, wv, outv, *sems):
    wid = lax.axis_index("s") * NC + lax.axis_index("c")
    base = wid * B_PER_W
    usems = sems[:NBUF]
    bsems = sems[NBUF:]

    pltpu.sync_copy(uidx_hbm.at[wid], uiv)
    pltpu.sync_copy(bidx_hbm.at[wid], biv)
    pltpu.sync_copy(wb_hbm, wv)

    wu0 = wv[pl.ds(0, L)]
    wu1 = wv[pl.ds(L, L)]
    wb0 = wv[pl.ds(2 * L, L)]
    wb1 = wv[pl.ds(3 * L, L)]
    bias16 = wv[pl.ds(4 * L, L)]
    d_lo = lax.iota(jnp.int32, L)
    d_hi = d_lo + L
    zero16 = jnp.zeros((L,), jnp.int32)

    def lane_of(k):
        start = (k // L) * L
        return start, k - start

    def col_scalar(iv, k):
        start, lane = lane_of(k)
        v = iv[pl.ds(start, L)]
        tcol = jnp.where(d_lo == lane, lax.shift_right_logical(v, 7), zero16)
        return pl.multiple_of(lax.reduce_max(tcol, axes=(0,)) * LANES, LANES)

    def lane_bcast(iv, k):
        start, lane = lane_of(k)
        v = iv[pl.ds(start, L)] & (LANES - 1)
        return v[jnp.full((L,), lane, jnp.int32)]

    def issue(k, p):
        cu = col_scalar(uiv, k)
        cb = col_scalar(biv, k)
        for a in range(4):
            rows = pl.ds(a * 8, 8)
            pltpu.async_copy(ut_hbm.at[rows, pl.ds(cu, LANES)],
                             ustg.at[p].at[rows], usems[p])
            pltpu.async_copy(bt_hbm.at[rows, pl.ds(cb, LANES)],
                             bstg.at[p].at[rows], bsems[p])

    def wait_slot(p):
        pltpu.make_async_copy(ut_hbm.at[:, pl.ds(0, LANES)],
                              ustg.at[p], usems[p]).wait()
        pltpu.make_async_copy(bt_hbm.at[:, pl.ds(0, LANES)],
                              bstg.at[p], bsems[p]).wait()

    def compute(j, p, vacc):
        clu = lane_bcast(uiv, j)
        clb = lane_bcast(biv, j)
        us = ustg.at[p]
        bs = bstg.at[p]
        acc = plsc.load_gather(us, [d_lo, clu]) * wu0 \
            + plsc.load_gather(us, [d_hi, clu]) * wu1 \
            + plsc.load_gather(bs, [d_lo, clb]) * wb0 \
            + plsc.load_gather(bs, [d_hi, clb]) * wb1 \
            + bias16
        s = lax.reduce_sum(acc, axes=(0,))
        _, lane = lane_of(j)
        return jnp.where(d_lo == lane, s, vacc)

    for k in range(NBUF):
        issue(k, k)

    def block(kb, vacc):
        for p in range(NBUF):
            k = kb * NBUF + p
            wait_slot(p)
            vacc = compute(k, p, vacc)

            @pl.when(k + NBUF < B_PER_W)
            def _():
                issue(k + NBUF, p)

        @pl.when(lax.rem(kb, L // NBUF) == L // NBUF - 1)
        def _():
            outv[pl.ds((kb // (L // NBUF)) * L, L)] = vacc
        return vacc

    lax.fori_loop(0, B_PER_W // NBUF, block, jnp.zeros((L,), jnp.float32))
    pltpu.sync_copy(outv, out_hbm.at[pl.ds(base, B_PER_W)])


def kernel(users, books, user_table, book_table, W, b):
    batch = users.shape[0]
    uidx = users.astype(jnp.int32).reshape(NW, B_PER_W)
    bidx = books.astype(jnp.int32).reshape(NW, B_PER_W)
    ut_t = user_table.T
    bt_t = book_table.T
    wb = jnp.concatenate(
        [W.reshape(2 * EMBED).astype(jnp.float32),
         jnp.broadcast_to(b.reshape(1) / L, (L,))])

    mesh = plsc.VectorSubcoreMesh(core_axis_name="c", subcore_axis_name="s")
    fn = functools.partial(
        pl.kernel,
        out_type=jax.ShapeDtypeStruct((batch,), jnp.float32),
        mesh=mesh,
        scratch_types=[
            pltpu.VMEM((B_PER_W,), jnp.int32),
            pltpu.VMEM((B_PER_W,), jnp.int32),
            pltpu.VMEM((NBUF, EMBED, LANES), jnp.float32),
            pltpu.VMEM((NBUF, EMBED, LANES), jnp.float32),
            pltpu.VMEM((WVLEN,), jnp.float32),
            pltpu.VMEM((B_PER_W,), jnp.float32),
        ] + [pltpu.SemaphoreType.DMA] * (2 * NBUF),
        compiler_params=pltpu.CompilerParams(
            needs_layout_passes=False, use_tc_tiling_on_sc=True),
    )(_sc_body)
    out = fn(uidx, bidx, ut_t, bt_t, wb)
    return out.reshape(batch, 1)


# speedup vs baseline: 1.3785x; 1.0189x over previous
import functools

import jax
import jax.numpy as jnp
from jax import lax
from jax.experimental import pallas as pl
from jax.experimental.pallas import tpu as pltpu
from jax.experimental.pallas import tpu_sc as plsc

NC = 2
NS = 16
NW = NC * NS
L = 16
EMBED = 32
LANES = 128
NBUF = 8
BATCH = 16384
B_PER_W = BATCH // NW
WVLEN = 4 * L + L


def _sc_body(uidx_hbm, bidx_hbm, ut_hbm, bt_hbm, wb_hbm, out_hbm,
             uiv, biv, ustg, bstg, wv, outv, *sems):
    wid = lax.axis_index("s") * NC + lax.axis_index("c")
    base = wid * B_PER_W
    usems = sems[:NBUF]
    bsems = sems[NBUF:]

    pltpu.sync_copy(uidx_hbm.at[wid], uiv)
    pltpu.sync_copy(bidx_hbm.at[wid], biv)
    pltpu.sync_copy(wb_hbm, wv)

    wu0 = wv[pl.ds(0, L)]
    wu1 = wv[pl.ds(L, L)]
    wb0 = wv[pl.ds(2 * L, L)]
    wb1 = wv[pl.ds(3 * L, L)]
    bias16 = wv[pl.ds(4 * L, L)]
    d_lo = lax.iota(jnp.int32, L)
    d_hi = d_lo + L
    zero16 = jnp.zeros((L,), jnp.int32)

    def lane_of(k):
        start = (k // L) * L
        return start, k - start

    def col_scalar(iv, k):
        start, lane = lane_of(k)
        v = iv[pl.ds(start, L)]
        tcol = jnp.where(d_lo == lane, lax.shift_right_logical(v, 7), zero16)
        return pl.multiple_of(lax.reduce_max(tcol, axes=(0,)) * LANES, LANES)

    def lane_bcast(iv, k):
        start, lane = lane_of(k)
        v = iv[pl.ds(start, L)] & (LANES - 1)
        return v[jnp.full((L,), lane, jnp.int32)]

    def issue(k, p):
        pltpu.async_copy(ut_hbm.at[:, pl.ds(col_scalar(uiv, k), LANES)],
                         ustg.at[p], usems[p])
        pltpu.async_copy(bt_hbm.at[:, pl.ds(col_scalar(biv, k), LANES)],
                         bstg.at[p], bsems[p])

    def wait_slot(p):
        pltpu.make_async_copy(ut_hbm.at[:, pl.ds(0, LANES)],
                              ustg.at[p], usems[p]).wait()
        pltpu.make_async_copy(bt_hbm.at[:, pl.ds(0, LANES)],
                              bstg.at[p], bsems[p]).wait()

    def compute(j, p, vacc):
        clu = lane_bcast(uiv, j)
        clb = lane_bcast(biv, j)
        us = ustg.at[p]
        bs = bstg.at[p]
        acc = plsc.load_gather(us, [d_lo, clu]) * wu0 \
            + plsc.load_gather(us, [d_hi, clu]) * wu1 \
            + plsc.load_gather(bs, [d_lo, clb]) * wb0 \
            + plsc.load_gather(bs, [d_hi, clb]) * wb1 \
            + bias16
        s = lax.reduce_sum(acc, axes=(0,))
        _, lane = lane_of(j)
        return jnp.where(d_lo == lane, s, vacc)

    for k in range(NBUF):
        issue(k, k)

    def block(kb, vacc):
        for p in range(NBUF):
            k = kb * NBUF + p
            wait_slot(p)
            vacc = compute(k, p, vacc)

            @pl.when(k + NBUF < B_PER_W)
            def _():
                issue(k + NBUF, p)

        @pl.when(lax.rem(kb, L // NBUF) == L // NBUF - 1)
        def _():
            outv[pl.ds((kb // (L // NBUF)) * L, L)] = vacc
        return vacc

    lax.fori_loop(0, B_PER_W // NBUF, block, jnp.zeros((L,), jnp.float32))
    pltpu.sync_copy(outv, out_hbm.at[pl.ds(base, B_PER_W)])


def kernel(users, books, user_table, book_table, W, b):
    batch = users.shape[0]
    uidx = users.astype(jnp.int32).reshape(NW, B_PER_W)
    bidx = books.astype(jnp.int32).reshape(NW, B_PER_W)
    ut_t = user_table.T
    bt_t = book_table.T
    wb = jnp.concatenate(
        [W.reshape(2 * EMBED).astype(jnp.float32),
         jnp.broadcast_to(b.reshape(1) / L, (L,))])

    mesh = plsc.VectorSubcoreMesh(core_axis_name="c", subcore_axis_name="s")
    fn = functools.partial(
        pl.kernel,
        out_type=jax.ShapeDtypeStruct((batch,), jnp.float32),
        mesh=mesh,
        scratch_types=[
            pltpu.VMEM((B_PER_W,), jnp.int32),
            pltpu.VMEM((B_PER_W,), jnp.int32),
            pltpu.VMEM((NBUF, EMBED, LANES), jnp.float32),
            pltpu.VMEM((NBUF, EMBED, LANES), jnp.float32),
            pltpu.VMEM((WVLEN,), jnp.float32),
            pltpu.VMEM((B_PER_W,), jnp.float32),
        ] + [pltpu.SemaphoreType.DMA] * (2 * NBUF),
        compiler_params=pltpu.CompilerParams(
            needs_layout_passes=False, use_tc_tiling_on_sc=True),
    )(_sc_body)
    out = fn(uidx, bidx, ut_t, bt_t, wb)
    return out.reshape(batch, 1)
